# Initial kernel scaffold; baseline (speedup 1.0000x reference)
import jax
import jax.numpy as jnp
from jax.experimental import pallas as pl


def kernel(x_paper, x_author, edge_index_pp, edge_index_ap, W_gcn1, b_gcn1, Wl_sage1, bl_sage1, Wr_sage1, W_gcn2, b_gcn2, Wl_sage2, bl_sage2, Wr_sage2):
    raise NotImplementedError("write your pallas kernel here")


# speedup vs baseline: 2.9162x; 2.9162x over previous
import jax
import jax.numpy as jnp
from jax import lax
from jax.experimental import pallas as pl
from jax.experimental.pallas import tpu as pltpu
from jax.experimental.pallas import tpu_sc as plsc

f32 = jnp.float32
i32 = jnp.int32

N = 50000
NP = 50176
E = 800000
EP = 802816
B = 128
RPW = 392
K = 8
G = RPW // K
RPT = NP // 16
ZR = 784
Q = 16
BLK = 1024
NB = NP // BLK

_mesh = plsc.VectorSubcoreMesh(core_axis_name="c", subcore_axis_name="s")


def _segsum_body(slo, dlo, shi, dhi, tlo, thi, olo, ohi,
                 sidx, didx, rows, zb, acc, sem):
    cid = lax.axis_index("c")
    sid = lax.axis_index("s")
    zeros16 = jnp.zeros((16,), f32)

    def _z(i, c):
        zb[i, pl.ds(0, 16)] = zeros16
        return c
    lax.fori_loop(0, ZR, _z, 0)
    base = sid * RPT
    for q in range(4):
        pltpu.sync_copy(zb, acc.at[pl.ds(base + q * ZR, ZR)])
    plsc.subcore_barrier()

    def _accum(src_hbm, dst_hbm, tbl):
        def body(g, c):
            r0 = sid * RPW + g * K
            pltpu.sync_copy(src_hbm.at[pl.ds(r0, K)], sidx)
            pltpu.sync_copy(dst_hbm.at[pl.ds(r0, K)], didx)
            cps = [pltpu.async_copy(tbl.at[sidx.at[j]], rows.at[j], sem)
                   for j in range(K)]
            for cp in cps:
                cp.wait()
            for j in range(K):
                pltpu.sync_copy(rows.at[j], acc.at[didx.at[j]], add=True)
            return c
        lax.fori_loop(0, G, body, 0)

    @pl.when(cid == 0)
    def _():
        _accum(slo, dlo, tlo)

    @pl.when(cid == 1)
    def _():
        _accum(shi, dhi, thi)

    plsc.subcore_barrier()

    def _wb(out):
        for q in range(4):
            pltpu.sync_copy(acc.at[pl.ds(base + q * ZR, ZR)], zb)
            pltpu.sync_copy(zb, out.at[pl.ds(base + q * ZR, ZR)])

    @pl.when(cid == 0)
    def _():
        _wb(olo)

    @pl.when(cid == 1)
    def _():
        _wb(ohi)


_segsum = pl.kernel(
    _segsum_body,
    out_type=[jax.ShapeDtypeStruct((NP, Q), f32)] * 2,
    mesh=_mesh,
    scratch_types=[
        pltpu.VMEM((K, B), i32),
        pltpu.VMEM((K, B), i32),
        pltpu.VMEM((K, B, Q), f32),
        pltpu.VMEM((ZR, Q), f32),
        pltpu.VMEM_SHARED((NP, Q), f32),
        pltpu.SemaphoreType.DMA,
    ],
    compiler_params=pltpu.CompilerParams(use_tc_tiling_on_sc=False),
)


def _row_spec(w):
    return pl.BlockSpec((BLK, w), lambda i: (i, 0))


def _full_spec(r, c):
    return pl.BlockSpec((r, c), lambda i: (0, 0))


def _t0_body(x_ref, q0, q1, q2, q3):
    x = x_ref[...]
    q0[...] = x[:, 0:16]
    q1[...] = x[:, 16:32]
    q2[...] = x[:, 32:48]
    q3[...] = x[:, 48:64]


_t0 = pl.pallas_call(
    _t0_body,
    grid=(NB,),
    in_specs=[_row_spec(64)],
    out_specs=[_row_spec(Q)] * 4,
    out_shape=[jax.ShapeDtypeStruct((NP, Q), f32)] * 4,
)


def _t1_body(x_ref, w_ref, deg_ref, q0, q1, q2, q3):
    dis = lax.rsqrt(deg_ref[:, 0:1] + 1.0)
    y = jnp.dot(x_ref[...], w_ref[...], preferred_element_type=f32) * dis
    q0[...] = y[:, 0:16]
    q1[...] = y[:, 16:32]
    q2[...] = y[:, 32:48]
    q3[...] = y[:, 48:64]


_t1 = pl.pallas_call(
    _t1_body,
    grid=(NB,),
    in_specs=[_row_spec(32), _full_spec(32, 64), _row_spec(Q)],
    out_specs=[_row_spec(Q)] * 4,
    out_shape=[jax.ShapeDtypeStruct((NP, Q), f32)] * 4,
)


def _t2_body(s0, s1, s2, s3, y0, y1, y2, y3, a0, a1, a2, a3,
             deg, cnt, x_ref, wl1, wr1, bs1, wg2,
             h0, h1, h2, h3, z0, z1, z2, z3):
    dis = lax.rsqrt(deg[:, 0:1] + 1.0)
    invc = 1.0 / jnp.maximum(cnt[:, 0:1], 1.0)
    aq = (a0, a1, a2, a3)
    dense = jnp.dot(x_ref[...], wr1[...], preferred_element_type=f32) + bs1[...]
    for q in range(4):
        dense += jnp.dot(aq[q][...] * invc, wl1[Q * q:Q * (q + 1), :],
                         preferred_element_type=f32)
    sq = (s0, s1, s2, s3)
    yq = (y0, y1, y2, y3)
    hq = [jnp.maximum(dis * (sq[q][...] + yq[q][...])
                      + dense[:, Q * q:Q * (q + 1)], 0.0) for q in range(4)]
    h0[...], h1[...], h2[...], h3[...] = hq
    y2f = jnp.dot(hq[0], wg2[0:Q, :], preferred_element_type=f32)
    for q in range(1, 4):
        y2f += jnp.dot(hq[q], wg2[Q * q:Q * (q + 1), :],
                       preferred_element_type=f32)
    y2f = y2f * dis
    z0[...] = y2f[:, 0:16]
    z1[...] = y2f[:, 16:32]
    z2[...] = y2f[:, 32:48]
    z3[...] = y2f[:, 48:64]


_t2 = pl.pallas_call(
    _t2_body,
    grid=(NB,),
    in_specs=[_row_spec(Q)] * 12 + [_row_spec(Q)] * 2 + [_row_spec(32),
              _full_spec(64, 64), _full_spec(32, 64), _full_spec(1, 64),
              _full_spec(64, 64)],
    out_specs=[_row_spec(Q)] * 8,
    out_shape=[jax.ShapeDtypeStruct((NP, Q), f32)] * 8,
)


def _t3_body(s0, s1, s2, s3, y0, y1, y2, y3, a0, a1, a2, a3,
             deg, cnt, h0, h1, h2, h3, wl2, wr2, bs2, out_ref):
    dis = lax.rsqrt(deg[:, 0:1] + 1.0)
    invc = 1.0 / jnp.maximum(cnt[:, 0:1], 1.0)
    aq = (a0, a1, a2, a3)
    hq = (h0, h1, h2, h3)
    dense = bs2[...] + jnp.zeros((BLK, 64), f32)
    for q in range(4):
        dense += jnp.dot(aq[q][...] * invc, wl2[Q * q:Q * (q + 1), :],
                         preferred_element_type=f32)
        dense += jnp.dot(hq[q][...], wr2[Q * q:Q * (q + 1), :],
                         preferred_element_type=f32)
    sq = (s0, s1, s2, s3)
    yq = (y0, y1, y2, y3)
    for q in range(4):
        out_ref[:, Q * q:Q * (q + 1)] = (
            dense[:, Q * q:Q * (q + 1)] + dis * (sq[q][...] + yq[q][...]))


_t3 = pl.pallas_call(
    _t3_body,
    grid=(NB,),
    in_specs=[_row_spec(Q)] * 12 + [_row_spec(Q)] * 2 + [_row_spec(Q)] * 4
             + [_full_spec(64, 64), _full_spec(64, 64), _full_spec(1, 64)],
    out_specs=_row_spec(64),
    out_shape=jax.ShapeDtypeStruct((N, 64), f32),
)


def _prep_edges(ei):
    epad = EP - E
    src = jnp.concatenate([ei[0], jnp.zeros((epad,), i32)]).reshape(EP // B, B)
    dst = jnp.concatenate([ei[1], jnp.full((epad,), NP - 1, i32)]
                          ).reshape(EP // B, B)
    return src, dst


def kernel(x_paper, x_author, edge_index_pp, edge_index_ap,
           W_gcn1, b_gcn1, Wl_sage1, bl_sage1, Wr_sage1,
           W_gcn2, b_gcn2, Wl_sage2, bl_sage2, Wr_sage2):
    xa = _t0(x_author)
    spp, dpp = _prep_edges(edge_index_pp)
    sap, dap = _prep_edges(edge_index_ap)

    ones_tbl = jnp.ones((NP, Q), f32)
    zsrc = jnp.zeros((EP // B, B), i32)
    deg, cnt = _segsum(zsrc, dpp, zsrc, dap, ones_tbl, ones_tbl)
    sap, dap, deg, cnt = lax.optimization_barrier((sap, dap, deg, cnt))
    a01 = _segsum(sap, dap, sap, dap, xa[0], xa[1])
    a23 = _segsum(sap, dap, sap, dap, xa[2], xa[3])
    aq = (a01[0], a01[1], a23[0], a23[1])
    y1 = _t1(x_paper, W_gcn1, deg)
    spp, dpp, aq3 = lax.optimization_barrier((spp, dpp, aq[3]))
    aq = (aq[0], aq[1], aq[2], aq3)
    s11 = _segsum(spp, dpp, spp, dpp, y1[0], y1[1])
    s12 = _segsum(spp, dpp, spp, dpp, y1[2], y1[3])

    bs1 = (b_gcn1 + bl_sage1).reshape(1, 64)
    bs2 = (b_gcn2 + bl_sage2).reshape(1, 64)
    t2out = _t2(s11[0], s11[1], s12[0], s12[1], y1[0], y1[1], y1[2], y1[3],
                aq[0], aq[1], aq[2], aq[3], deg, cnt, x_paper,
                Wl_sage1, Wr_sage1, bs1, W_gcn2)
    hq = t2out[0:4]
    y2 = t2out[4:8]
    s21 = _segsum(spp, dpp, spp, dpp, y2[0], y2[1])
    s22 = _segsum(spp, dpp, spp, dpp, y2[2], y2[3])
    return _t3(s21[0], s21[1], s22[0], s22[1], y2[0], y2[1], y2[2], y2[3],
               aq[0], aq[1], aq[2], aq[3], deg, cnt,
               hq[0], hq[1], hq[2], hq[3], Wl_sage2, Wr_sage2, bs2)


# speedup vs baseline: 12.5968x; 4.3196x over previous
import jax
import jax.numpy as jnp
from jax import lax
from jax.experimental import pallas as pl
from jax.experimental.pallas import tpu as pltpu
from jax.experimental.pallas import tpu_sc as plsc

f32 = jnp.float32
i32 = jnp.int32

N = 50000
NP = 50176
E = 800000
EP = 802816
B = 128
RPW = 392
K = 8
G = RPW // K
RPT = NP // 16
ZR = 784
Q = 16
BLK = 1024
NB = NP // BLK

_mesh = plsc.VectorSubcoreMesh(core_axis_name="c", subcore_axis_name="s")


def _segsum_body(slo, dlo, shi, dhi, tlo, thi, olo, ohi,
                 sidx, didx, rows, zb, acc, sem):
    cid = lax.axis_index("c")
    sid = lax.axis_index("s")
    zeros16 = jnp.zeros((16,), f32)

    def _z(i, c):
        zb[i, pl.ds(0, 16)] = zeros16
        return c
    lax.fori_loop(0, ZR, _z, 0)
    base = sid * RPT
    for q in range(4):
        pltpu.sync_copy(zb, acc.at[pl.ds(base + q * ZR, ZR)])
    plsc.subcore_barrier()

    def _accum(src_hbm, dst_hbm, tbl):
        def body(g, c):
            r0 = sid * RPW + g * K
            pltpu.sync_copy(src_hbm.at[pl.ds(r0, K)], sidx)
            pltpu.sync_copy(dst_hbm.at[pl.ds(r0, K)], didx)
            cps = [pltpu.async_copy(tbl.at[sidx.at[j]], rows.at[j], sem)
                   for j in range(K)]
            for cp in cps:
                cp.wait()
            for j in range(K):
                pltpu.sync_copy(rows.at[j], acc.at[didx.at[j]], add=True)
            return c
        lax.fori_loop(0, G, body, 0)

    @pl.when(cid == 0)
    def _():
        _accum(slo, dlo, tlo)

    @pl.when(cid == 1)
    def _():
        _accum(shi, dhi, thi)

    plsc.subcore_barrier()

    def _wb(out):
        for q in range(4):
            pltpu.sync_copy(acc.at[pl.ds(base + q * ZR, ZR)], zb)
            pltpu.sync_copy(zb, out.at[pl.ds(base + q * ZR, ZR)])

    @pl.when(cid == 0)
    def _():
        _wb(olo)

    @pl.when(cid == 1)
    def _():
        _wb(ohi)


_segsum = pl.kernel(
    _segsum_body,
    out_type=[jax.ShapeDtypeStruct((NP, Q), f32)] * 2,
    mesh=_mesh,
    scratch_types=[
        pltpu.VMEM((K, B), i32),
        pltpu.VMEM((K, B), i32),
        pltpu.VMEM((K, B, Q), f32),
        pltpu.VMEM((ZR, Q), f32),
        pltpu.VMEM_SHARED((NP, Q), f32),
        pltpu.SemaphoreType.DMA,
    ],
    compiler_params=pltpu.CompilerParams(use_tc_tiling_on_sc=False),
)


def _row_spec(w):
    return pl.BlockSpec((BLK, w), lambda i: (i, 0))


def _full_spec(r, c):
    return pl.BlockSpec((r, c), lambda i: (0, 0))


def _t0_body(x_ref, q0, q1, q2, q3):
    x = x_ref[...]
    q0[...] = x[:, 0:16]
    q1[...] = x[:, 16:32]
    q2[...] = x[:, 32:48]
    q3[...] = x[:, 48:64]


_t0 = pl.pallas_call(
    _t0_body,
    grid=(NB,),
    in_specs=[_row_spec(64)],
    out_specs=[_row_spec(Q)] * 4,
    out_shape=[jax.ShapeDtypeStruct((NP, Q), f32)] * 4,
)


def _t1_body(x_ref, w_ref, deg_ref, q0, q1, q2, q3):
    dis = lax.rsqrt(deg_ref[:, 0:1] + 1.0)
    y = jnp.dot(x_ref[...], w_ref[...], preferred_element_type=f32) * dis
    q0[...] = y[:, 0:16]
    q1[...] = y[:, 16:32]
    q2[...] = y[:, 32:48]
    q3[...] = y[:, 48:64]


_t1 = pl.pallas_call(
    _t1_body,
    grid=(NB,),
    in_specs=[_row_spec(32), _full_spec(32, 64), _row_spec(Q)],
    out_specs=[_row_spec(Q)] * 4,
    out_shape=[jax.ShapeDtypeStruct((NP, Q), f32)] * 4,
)


def _t2_body(s0, s1, s2, s3, y0, y1, y2, y3, a0, a1, a2, a3,
             deg, cnt, x_ref, wl1, wr1, bs1, wg2,
             h0, h1, h2, h3, z0, z1, z2, z3):
    dis = lax.rsqrt(deg[:, 0:1] + 1.0)
    invc = 1.0 / jnp.maximum(cnt[:, 0:1], 1.0)
    aq = (a0, a1, a2, a3)
    dense = jnp.dot(x_ref[...], wr1[...], preferred_element_type=f32) + bs1[...]
    for q in range(4):
        dense += jnp.dot(aq[q][...] * invc, wl1[Q * q:Q * (q + 1), :],
                         preferred_element_type=f32)
    sq = (s0, s1, s2, s3)
    yq = (y0, y1, y2, y3)
    hq = [jnp.maximum(dis * (sq[q][...] + yq[q][...])
                      + dense[:, Q * q:Q * (q + 1)], 0.0) for q in range(4)]
    h0[...], h1[...], h2[...], h3[...] = hq
    y2f = jnp.dot(hq[0], wg2[0:Q, :], preferred_element_type=f32)
    for q in range(1, 4):
        y2f += jnp.dot(hq[q], wg2[Q * q:Q * (q + 1), :],
                       preferred_element_type=f32)
    y2f = y2f * dis
    z0[...] = y2f[:, 0:16]
    z1[...] = y2f[:, 16:32]
    z2[...] = y2f[:, 32:48]
    z3[...] = y2f[:, 48:64]


_t2 = pl.pallas_call(
    _t2_body,
    grid=(NB,),
    in_specs=[_row_spec(Q)] * 12 + [_row_spec(Q)] * 2 + [_row_spec(32),
              _full_spec(64, 64), _full_spec(32, 64), _full_spec(1, 64),
              _full_spec(64, 64)],
    out_specs=[_row_spec(Q)] * 8,
    out_shape=[jax.ShapeDtypeStruct((NP, Q), f32)] * 8,
)


def _t3_body(s0, s1, s2, s3, y0, y1, y2, y3, a0, a1, a2, a3,
             deg, cnt, h0, h1, h2, h3, wl2, wr2, bs2, out_ref):
    dis = lax.rsqrt(deg[:, 0:1] + 1.0)
    invc = 1.0 / jnp.maximum(cnt[:, 0:1], 1.0)
    aq = (a0, a1, a2, a3)
    hq = (h0, h1, h2, h3)
    dense = bs2[...] + jnp.zeros((BLK, 64), f32)
    for q in range(4):
        dense += jnp.dot(aq[q][...] * invc, wl2[Q * q:Q * (q + 1), :],
                         preferred_element_type=f32)
        dense += jnp.dot(hq[q][...], wr2[Q * q:Q * (q + 1), :],
                         preferred_element_type=f32)
    sq = (s0, s1, s2, s3)
    yq = (y0, y1, y2, y3)
    for q in range(4):
        out_ref[:, Q * q:Q * (q + 1)] = (
            dense[:, Q * q:Q * (q + 1)] + dis * (sq[q][...] + yq[q][...]))


_t3 = pl.pallas_call(
    _t3_body,
    grid=(NB,),
    in_specs=[_row_spec(Q)] * 12 + [_row_spec(Q)] * 2 + [_row_spec(Q)] * 4
             + [_full_spec(64, 64), _full_spec(64, 64), _full_spec(1, 64)],
    out_specs=_row_spec(64),
    out_shape=jax.ShapeDtypeStruct((N, 64), f32),
)


def _prep_edges(ei):
    epad = EP - E
    src = jnp.concatenate([ei[0], jnp.zeros((epad,), i32)]).reshape(EP // B, B)
    dst = jnp.concatenate([ei[1], jnp.full((epad,), NP - 1, i32)]
                          ).reshape(EP // B, B)
    return src, dst


def kernel(x_paper, x_author, edge_index_pp, edge_index_ap,
           W_gcn1, b_gcn1, Wl_sage1, bl_sage1, Wr_sage1,
           W_gcn2, b_gcn2, Wl_sage2, bl_sage2, Wr_sage2):
    xa = _t0(x_author)
    spp, dpp = _prep_edges(edge_index_pp)
    sap, dap = _prep_edges(edge_index_ap)

    ones_tbl = jnp.ones((NP, Q), f32)
    deg, cnt = _segsum(spp, dpp, sap, dap, ones_tbl, ones_tbl)
    sap, dap, deg, cnt = lax.optimization_barrier((sap, dap, deg, cnt))
    a01 = _segsum(sap, dap, sap, dap, xa[0], xa[1])
    a23 = _segsum(sap, dap, sap, dap, xa[2], xa[3])
    aq = (a01[0], a01[1], a23[0], a23[1])
    y1 = _t1(x_paper, W_gcn1, deg)
    spp, dpp, aq3 = lax.optimization_barrier((spp, dpp, aq[3]))
    aq = (aq[0], aq[1], aq[2], aq3)
    s11 = _segsum(spp, dpp, spp, dpp, y1[0], y1[1])
    s12 = _segsum(spp, dpp, spp, dpp, y1[2], y1[3])

    bs1 = (b_gcn1 + bl_sage1).reshape(1, 64)
    bs2 = (b_gcn2 + bl_sage2).reshape(1, 64)
    t2out = _t2(s11[0], s11[1], s12[0], s12[1], y1[0], y1[1], y1[2], y1[3],
                aq[0], aq[1], aq[2], aq[3], deg, cnt, x_paper,
                Wl_sage1, Wr_sage1, bs1, W_gcn2)
    hq = t2out[0:4]
    y2 = t2out[4:8]
    s21 = _segsum(spp, dpp, spp, dpp, y2[0], y2[1])
    s22 = _segsum(spp, dpp, spp, dpp, y2[2], y2[3])
    return _t3(s21[0], s21[1], s22[0], s22[1], y2[0], y2[1], y2[2], y2[3],
               aq[0], aq[1], aq[2], aq[3], deg, cnt,
               hq[0], hq[1], hq[2], hq[3], Wl_sage2, Wr_sage2, bs2)


# speedup vs baseline: 14.3986x; 1.1430x over previous
import jax
import jax.numpy as jnp
from jax import lax
from jax.experimental import pallas as pl
from jax.experimental.pallas import tpu as pltpu
from jax.experimental.pallas import tpu_sc as plsc

f32 = jnp.float32
i32 = jnp.int32

N = 50000
NP = 50176
E = 800000
EP = 802816
B = 128
RPW = 392
K = 4
G = RPW // K
NPAIR = G // 2
RPT = NP // 16
ZR = 784
Q = 16
BLK = 1024
NB = NP // BLK

_mesh = plsc.VectorSubcoreMesh(core_axis_name="c", subcore_axis_name="s")


def _segsum_body(slo, dlo, shi, dhi, tlo, thi, olo, ohi,
                 sidx, didx, rows, zb, acc, sem0, sem1):
    cid = lax.axis_index("c")
    sid = lax.axis_index("s")
    zeros16 = jnp.zeros((16,), f32)

    def _z(i, c):
        zb[i, pl.ds(0, 16)] = zeros16
        return c
    lax.fori_loop(0, ZR, _z, 0)
    base = sid * RPT
    for q in range(4):
        pltpu.sync_copy(zb, acc.at[pl.ds(base + q * ZR, ZR)])
    plsc.subcore_barrier()

    def _accum(src_hbm, dst_hbm, tbl):
        sems = (sem0, sem1)

        def idx_copy(g, slot):
            r0 = sid * RPW + g * K
            pltpu.sync_copy(src_hbm.at[pl.ds(r0, K)], sidx.at[slot])
            pltpu.sync_copy(dst_hbm.at[pl.ds(r0, K)], didx.at[slot])

        def fire(slot):
            for j in range(K):
                pltpu.async_copy(tbl.at[sidx.at[slot, j]],
                                 rows.at[slot, j], sems[slot])

        def drain(slot):
            for j in range(K):
                pltpu.make_async_copy(tbl.at[sidx.at[slot, j]],
                                      rows.at[slot, j], sems[slot]).wait()

        def adds(slot):
            for j in range(K):
                pltpu.sync_copy(rows.at[slot, j], acc.at[didx.at[slot, j]],
                                add=True)

        idx_copy(0, 0)
        fire(0)
        idx_copy(1, 1)

        def body(t, c):
            g2 = 2 * t + 2
            fire(1)
            drain(0)
            adds(0)

            @pl.when(g2 < G)
            def _():
                idx_copy(g2, 0)
                fire(0)

            drain(1)
            adds(1)

            @pl.when(g2 < G)
            def _():
                idx_copy(g2 + 1, 1)
            return c
        lax.fori_loop(0, NPAIR, body, 0)

    @pl.when(cid == 0)
    def _():
        _accum(slo, dlo, tlo)

    @pl.when(cid == 1)
    def _():
        _accum(shi, dhi, thi)

    plsc.subcore_barrier()

    def _wb(out):
        for q in range(4):
            pltpu.sync_copy(acc.at[pl.ds(base + q * ZR, ZR)], zb)
            pltpu.sync_copy(zb, out.at[pl.ds(base + q * ZR, ZR)])

    @pl.when(cid == 0)
    def _():
        _wb(olo)

    @pl.when(cid == 1)
    def _():
        _wb(ohi)


_segsum = pl.kernel(
    _segsum_body,
    out_type=[jax.ShapeDtypeStruct((NP, Q), f32)] * 2,
    mesh=_mesh,
    scratch_types=[
        pltpu.VMEM((2, K, B), i32),
        pltpu.VMEM((2, K, B), i32),
        pltpu.VMEM((2, K, B, Q), f32),
        pltpu.VMEM((ZR, Q), f32),
        pltpu.VMEM_SHARED((NP, Q), f32),
        pltpu.SemaphoreType.DMA,
        pltpu.SemaphoreType.DMA,
    ],
    compiler_params=pltpu.CompilerParams(use_tc_tiling_on_sc=False),
)


def _row_spec(w):
    return pl.BlockSpec((BLK, w), lambda i: (i, 0))


def _full_spec(r, c):
    return pl.BlockSpec((r, c), lambda i: (0, 0))


def _t0_body(x_ref, q0, q1, q2, q3):
    x = x_ref[...]
    q0[...] = x[:, 0:16]
    q1[...] = x[:, 16:32]
    q2[...] = x[:, 32:48]
    q3[...] = x[:, 48:64]


_t0 = pl.pallas_call(
    _t0_body,
    grid=(NB,),
    in_specs=[_row_spec(64)],
    out_specs=[_row_spec(Q)] * 4,
    out_shape=[jax.ShapeDtypeStruct((NP, Q), f32)] * 4,
)


def _t1_body(x_ref, w_ref, deg_ref, q0, q1, q2, q3):
    dis = lax.rsqrt(deg_ref[:, 0:1] + 1.0)
    y = jnp.dot(x_ref[...], w_ref[...], preferred_element_type=f32) * dis
    q0[...] = y[:, 0:16]
    q1[...] = y[:, 16:32]
    q2[...] = y[:, 32:48]
    q3[...] = y[:, 48:64]


_t1 = pl.pallas_call(
    _t1_body,
    grid=(NB,),
    in_specs=[_row_spec(32), _full_spec(32, 64), _row_spec(Q)],
    out_specs=[_row_spec(Q)] * 4,
    out_shape=[jax.ShapeDtypeStruct((NP, Q), f32)] * 4,
)


def _t2_body(s0, s1, s2, s3, y0, y1, y2, y3, a0, a1, a2, a3,
             deg, cnt, x_ref, wl1, wr1, bs1, wg2,
             h0, h1, h2, h3, z0, z1, z2, z3):
    dis = lax.rsqrt(deg[:, 0:1] + 1.0)
    invc = 1.0 / jnp.maximum(cnt[:, 0:1], 1.0)
    aq = (a0, a1, a2, a3)
    dense = jnp.dot(x_ref[...], wr1[...], preferred_element_type=f32) + bs1[...]
    for q in range(4):
        dense += jnp.dot(aq[q][...] * invc, wl1[Q * q:Q * (q + 1), :],
                         preferred_element_type=f32)
    sq = (s0, s1, s2, s3)
    yq = (y0, y1, y2, y3)
    hq = [jnp.maximum(dis * (sq[q][...] + yq[q][...])
                      + dense[:, Q * q:Q * (q + 1)], 0.0) for q in range(4)]
    h0[...], h1[...], h2[...], h3[...] = hq
    y2f = jnp.dot(hq[0], wg2[0:Q, :], preferred_element_type=f32)
    for q in range(1, 4):
        y2f += jnp.dot(hq[q], wg2[Q * q:Q * (q + 1), :],
                       preferred_element_type=f32)
    y2f = y2f * dis
    z0[...] = y2f[:, 0:16]
    z1[...] = y2f[:, 16:32]
    z2[...] = y2f[:, 32:48]
    z3[...] = y2f[:, 48:64]


_t2 = pl.pallas_call(
    _t2_body,
    grid=(NB,),
    in_specs=[_row_spec(Q)] * 12 + [_row_spec(Q)] * 2 + [_row_spec(32),
              _full_spec(64, 64), _full_spec(32, 64), _full_spec(1, 64),
              _full_spec(64, 64)],
    out_specs=[_row_spec(Q)] * 8,
    out_shape=[jax.ShapeDtypeStruct((NP, Q), f32)] * 8,
)


def _t3_body(s0, s1, s2, s3, y0, y1, y2, y3, a0, a1, a2, a3,
             deg, cnt, h0, h1, h2, h3, wl2, wr2, bs2, out_ref):
    dis = lax.rsqrt(deg[:, 0:1] + 1.0)
    invc = 1.0 / jnp.maximum(cnt[:, 0:1], 1.0)
    aq = (a0, a1, a2, a3)
    hq = (h0, h1, h2, h3)
    dense = bs2[...] + jnp.zeros((BLK, 64), f32)
    for q in range(4):
        dense += jnp.dot(aq[q][...] * invc, wl2[Q * q:Q * (q + 1), :],
                         preferred_element_type=f32)
        dense += jnp.dot(hq[q][...], wr2[Q * q:Q * (q + 1), :],
                         preferred_element_type=f32)
    sq = (s0, s1, s2, s3)
    yq = (y0, y1, y2, y3)
    for q in range(4):
        out_ref[:, Q * q:Q * (q + 1)] = (
            dense[:, Q * q:Q * (q + 1)] + dis * (sq[q][...] + yq[q][...]))


_t3 = pl.pallas_call(
    _t3_body,
    grid=(NB,),
    in_specs=[_row_spec(Q)] * 12 + [_row_spec(Q)] * 2 + [_row_spec(Q)] * 4
             + [_full_spec(64, 64), _full_spec(64, 64), _full_spec(1, 64)],
    out_specs=_row_spec(64),
    out_shape=jax.ShapeDtypeStruct((N, 64), f32),
)


def _prep_edges(ei):
    epad = EP - E
    src = jnp.concatenate([ei[0], jnp.zeros((epad,), i32)]).reshape(EP // B, B)
    dst = jnp.concatenate([ei[1], jnp.full((epad,), NP - 1, i32)]
                          ).reshape(EP // B, B)
    return src, dst


def kernel(x_paper, x_author, edge_index_pp, edge_index_ap,
           W_gcn1, b_gcn1, Wl_sage1, bl_sage1, Wr_sage1,
           W_gcn2, b_gcn2, Wl_sage2, bl_sage2, Wr_sage2):
    xa = _t0(x_author)
    spp, dpp = _prep_edges(edge_index_pp)
    sap, dap = _prep_edges(edge_index_ap)

    ones_tbl = jnp.ones((NP, Q), f32)
    deg, cnt = _segsum(spp, dpp, sap, dap, ones_tbl, ones_tbl)
    sap, dap, deg, cnt = lax.optimization_barrier((sap, dap, deg, cnt))
    a01 = _segsum(sap, dap, sap, dap, xa[0], xa[1])
    a23 = _segsum(sap, dap, sap, dap, xa[2], xa[3])
    aq = (a01[0], a01[1], a23[0], a23[1])
    y1 = _t1(x_paper, W_gcn1, deg)
    spp, dpp, aq3 = lax.optimization_barrier((spp, dpp, aq[3]))
    aq = (aq[0], aq[1], aq[2], aq3)
    s11 = _segsum(spp, dpp, spp, dpp, y1[0], y1[1])
    s12 = _segsum(spp, dpp, spp, dpp, y1[2], y1[3])

    bs1 = (b_gcn1 + bl_sage1).reshape(1, 64)
    bs2 = (b_gcn2 + bl_sage2).reshape(1, 64)
    t2out = _t2(s11[0], s11[1], s12[0], s12[1], y1[0], y1[1], y1[2], y1[3],
                aq[0], aq[1], aq[2], aq[3], deg, cnt, x_paper,
                Wl_sage1, Wr_sage1, bs1, W_gcn2)
    hq = t2out[0:4]
    y2 = t2out[4:8]
    s21 = _segsum(spp, dpp, spp, dpp, y2[0], y2[1])
    s22 = _segsum(spp, dpp, spp, dpp, y2[2], y2[3])
    return _t3(s21[0], s21[1], s22[0], s22[1], y2[0], y2[1], y2[2], y2[3],
               aq[0], aq[1], aq[2], aq[3], deg, cnt,
               hq[0], hq[1], hq[2], hq[3], Wl_sage2, Wr_sage2, bs2)
